# Initial kernel scaffold; baseline (speedup 1.0000x reference)
import jax
import jax.numpy as jnp
from jax.experimental import pallas as pl


def kernel(x, weights, bias):
    raise NotImplementedError("write your pallas kernel here")


# speedup vs baseline: 1.0088x; 1.0088x over previous
import functools

import jax
import jax.numpy as jnp
from jax import lax
from jax.experimental import pallas as pl
from jax.experimental.pallas import tpu as pltpu
from jax.experimental.pallas import tpu_sc as plsc

UNITS = 26
LOWER = -17.0
UPPER = 8.0
BW = 0.05
NUM_BUCKETS = int((UPPER - LOWER) / BW) + 1
RESIDUE = LOWER - BW
BATCH = 4096

UP = 32
KP = 512
NC = 2
NS = 16
NW = NC * NS
LANES = 16

TOTAL = BATCH * UNITS
PER_TILE = TOTAL // NW
VECS = PER_TILE // LANES
PAT_VECS = 13
OUTER = VECS // PAT_VECS

assert PER_TILE * NW == TOTAL and VECS * LANES == PER_TILE
assert PER_TILE % UNITS == 0
assert OUTER * PAT_VECS == VECS
assert PER_TILE % 8 == 0


def _tables_body(w_ref, b_ref, a_ref, r_ref, p_ref):
    r = jnp.maximum(w_ref[...], jnp.float32(0.0))
    ki = lax.broadcasted_iota(jnp.int32, (KP, KP), 0)
    ji = lax.broadcasted_iota(jnp.int32, (KP, KP), 1)
    tri = jnp.where(ki < ji, jnp.float32(BW), jnp.float32(0.0))
    acc = lax.dot(r, tri, precision=lax.Precision.HIGHEST,
                  preferred_element_type=jnp.float32)
    a_ref[...] = acc + jnp.float32(RESIDUE) + b_ref[...]
    r_ref[...] = r
    e = (lax.broadcasted_iota(jnp.int32, (LANES, LANES), 0) * LANES
         + lax.broadcasted_iota(jnp.int32, (LANES, LANES), 1))
    p_ref[...] = (e % UNITS) * KP


def _sc_body(x_hbm, a_hbm, r_hbm, p_hbm, o_hbm, x_v, a_v, r_v, p_v, o_v, sem):
    wid = lax.axis_index("s") * NC + lax.axis_index("c")
    base = wid * PER_TILE
    cx = pltpu.async_copy(x_hbm.at[pl.ds(base, PER_TILE)], x_v, sem)
    ca = pltpu.async_copy(a_hbm, a_v, sem)
    cr = pltpu.async_copy(r_hbm, r_v, sem)
    cp = pltpu.async_copy(p_hbm, p_v, sem)
    cx.wait()
    ca.wait()
    cr.wait()
    cp.wait()

    clo = jnp.float32(LOWER + 1e-09)
    chi = jnp.float32(UPPER - 1e-09)
    bwf = jnp.float32(BW)
    lowf = jnp.float32(LOWER)
    one = jnp.float32(1.0)

    def outer(o, carry):
        for j in range(PAT_VECS):
            off = o * (PAT_VECS * LANES) + j * LANES
            xc = jnp.clip(x_v[pl.ds(off, LANES)], clo, chi)
            s = xc - lowf + bwf
            idx = jnp.clip((s / bwf).astype(jnp.int32), 0, NUM_BUCKETS - 1)
            delta = s - idx.astype(jnp.float32) * bwf
            fl = p_v[pl.ds(j * LANES, LANES)] + idx
            av = plsc.load_gather(a_v, [fl])
            rv = plsc.load_gather(r_v, [fl])
            z = av + delta * rv
            o_v[pl.ds(off, LANES)] = one / (one + jnp.exp(-z))
        return carry

    lax.fori_loop(0, OUTER, outer, 0)
    pltpu.sync_copy(o_v, o_hbm.at[pl.ds(base, PER_TILE)])


def kernel(x, weights, bias):
    wp = jnp.pad(weights.astype(jnp.float32),
                 ((0, UP - UNITS), (0, KP - NUM_BUCKETS)))
    bp = jnp.pad(bias.astype(jnp.float32), (0, UP - UNITS)).reshape(UP, 1)
    a2, r2, p2 = pl.pallas_call(
        _tables_body,
        out_shape=[
            jax.ShapeDtypeStruct((UP, KP), jnp.float32),
            jax.ShapeDtypeStruct((UP, KP), jnp.float32),
            jax.ShapeDtypeStruct((LANES, LANES), jnp.int32),
        ],
    )(wp, bp)

    sc = functools.partial(
        pl.kernel,
        out_type=jax.ShapeDtypeStruct((TOTAL,), jnp.float32),
        mesh=plsc.VectorSubcoreMesh(core_axis_name="c", subcore_axis_name="s"),
        scratch_types=[
            pltpu.VMEM((PER_TILE,), jnp.float32),
            pltpu.VMEM((UP * KP,), jnp.float32),
            pltpu.VMEM((UP * KP,), jnp.float32),
            pltpu.VMEM((LANES * LANES,), jnp.int32),
            pltpu.VMEM((PER_TILE,), jnp.float32),
            pltpu.SemaphoreType.DMA,
        ],
        compiler_params=pltpu.CompilerParams(needs_layout_passes=False),
    )(_sc_body)
    out = sc(x.reshape(TOTAL), a2.reshape(UP * KP), r2.reshape(UP * KP),
             p2.reshape(LANES * LANES))
    return out.reshape(x.shape)


# speedup vs baseline: 1.6952x; 1.6804x over previous
import functools

import jax
import jax.numpy as jnp
from jax import lax
from jax.experimental import pallas as pl
from jax.experimental.pallas import tpu as pltpu
from jax.experimental.pallas import tpu_sc as plsc

UNITS = 26
LOWER = -17.0
UPPER = 8.0
BW = 0.05
NUM_BUCKETS = int((UPPER - LOWER) / BW) + 1
RESIDUE = LOWER - BW
BATCH = 4096

NC = 2
NS = 16
NW = NC * NS
LANES = 16

ROWS_PER_TILE = BATCH // NW
GROUPS = ROWS_PER_TILE // LANES
SUBROWS = 4
FULL_CHUNKS = NUM_BUCKETS // LANES
LAST_BASE = FULL_CHUNKS * LANES
LAST_N = NUM_BUCKETS - LAST_BASE
CPR = 128 // LANES

assert ROWS_PER_TILE * NW == BATCH and GROUPS * LANES == ROWS_PER_TILE


def _sc_body(x_hbm, w_hbm, b_hbm, o_hbm,
             x_v, w_v, b_v, a_v, c_v, o_v, carry_v, sem):
    wid = lax.axis_index("s") * NC + lax.axis_index("c")
    col0 = wid * ROWS_PER_TILE
    cx = pltpu.async_copy(x_hbm.at[:, pl.ds(col0, ROWS_PER_TILE)], x_v, sem)
    cw = pltpu.async_copy(w_hbm, w_v, sem)
    cb = pltpu.async_copy(b_hbm, b_v, sem)
    cx.wait()

    clo = jnp.float32(LOWER + 1e-09)
    chi = jnp.float32(UPPER - 1e-09)
    bwf = jnp.float32(BW)
    resf = jnp.float32(RESIDUE)
    shiftf = jnp.float32(BW - LOWER)
    invbw = jnp.float32(1.0 / BW)
    idxcap = jnp.float32(NUM_BUCKETS - 0.5)
    half = jnp.float32(0.5)
    two = jnp.float32(2.0)
    one = jnp.float32(1.0)
    zero = jnp.float32(0.0)
    lane = lax.iota(jnp.int32, LANES)

    @plsc.parallel_loop(0, UNITS * GROUPS, unroll=8)
    def bucketize(i):
        u = i >> 3
        off = (i & (GROUPS - 1)) * LANES
        xc = jnp.clip(x_v.at[u][pl.ds(off, LANES)], clo, chi)
        t = (xc + shiftf) * invbw
        idxf = jnp.minimum(t, idxcap).astype(jnp.int32).astype(jnp.float32)
        c_v.at[u][pl.ds(off, LANES)] = idxf + (t - idxf) * half

    cw.wait()
    cb.wait()

    @plsc.parallel_loop(0, UNITS, unroll=2)
    def carry_init(u):
        ucol = jnp.full((LANES,), u, jnp.int32)
        bias_u = plsc.load_gather(b_v, [jnp.zeros((LANES,), jnp.int32), ucol])
        carry_v.at[u][pl.ds(0, LANES)] = bias_u + resf

    def chunk_step(m, _):
        sub = m // CPR
        coff = (m % CPR) * LANES

        @plsc.parallel_loop(0, UNITS, unroll=2)
        def row(u):
            ucol = jnp.full((LANES,), u, jnp.int32)
            g = plsc.load_gather(w_v, [ucol, m * LANES + lane])
            w16 = jnp.maximum(g, zero)
            inc = plsc.cumsum(w16)
            cval = carry_v.at[u][pl.ds(0, LANES)]
            a_v.at[u * SUBROWS + sub][pl.ds(coff, LANES)] = (
                (inc - w16) * bwf + cval)
            carry_v.at[u][pl.ds(0, LANES)] = cval + jnp.sum(w16) * bwf
        return 0

    lax.fori_loop(0, FULL_CHUNKS, chunk_step, 0)

    @plsc.parallel_loop(0, UNITS, unroll=2)
    def last_row(u):
        ucol = jnp.full((LANES,), u, jnp.int32)
        idxs = jnp.minimum(LAST_BASE + lane, NUM_BUCKETS - 1)
        g = plsc.load_gather(w_v, [ucol, idxs])
        w16 = jnp.where(lane < LAST_N, jnp.maximum(g, zero), zero)
        inc = plsc.cumsum(w16)
        cval = carry_v.at[u][pl.ds(0, LANES)]
        a_v.at[u * SUBROWS + 3][pl.ds(LAST_BASE - 384, LANES)] = (
            (inc - w16) * bwf + cval)

    @plsc.parallel_loop(0, UNITS * GROUPS, unroll=8)
    def combine(i):
        u = i >> 3
        off = (i & (GROUPS - 1)) * LANES
        u4 = jnp.full((LANES,), u * SUBROWS, jnp.int32)
        c = c_v.at[u][pl.ds(off, LANES)]
        idx = c.astype(jnp.int32)
        frac2 = c - idx.astype(jnp.float32)
        idx1 = idx + 1
        a0 = plsc.load_gather(a_v, [u4 + (idx >> 7), idx & 127])
        a1 = plsc.load_gather(a_v, [u4 + (idx1 >> 7), idx1 & 127])
        z = a0 + frac2 * two * (a1 - a0)
        o_v.at[u][pl.ds(off, LANES)] = one / (one + jnp.exp(-z))

    pltpu.sync_copy(o_v, o_hbm.at[:, pl.ds(col0, ROWS_PER_TILE)])


def kernel(x, weights, bias):
    sc = functools.partial(
        pl.kernel,
        out_type=jax.ShapeDtypeStruct((UNITS, BATCH), jnp.float32),
        mesh=plsc.VectorSubcoreMesh(core_axis_name="c", subcore_axis_name="s"),
        scratch_types=[
            pltpu.VMEM((UNITS, ROWS_PER_TILE), jnp.float32),
            pltpu.VMEM((UNITS, NUM_BUCKETS), jnp.float32),
            pltpu.VMEM((1, UNITS), jnp.float32),
            pltpu.VMEM((UNITS * SUBROWS, 128), jnp.float32),
            pltpu.VMEM((UNITS, ROWS_PER_TILE), jnp.float32),
            pltpu.VMEM((UNITS, ROWS_PER_TILE), jnp.float32),
            pltpu.VMEM((UNITS, 128), jnp.float32),
            pltpu.SemaphoreType.DMA,
        ],
        compiler_params=pltpu.CompilerParams(needs_layout_passes=False,
                                             use_tc_tiling_on_sc=True,
                                             skip_device_barrier=True),
    )(_sc_body)
    return sc(x.T, weights.astype(jnp.float32),
              bias.astype(jnp.float32)[None, :]).T


# speedup vs baseline: 1.9432x; 1.1463x over previous
import functools

import jax
import jax.numpy as jnp
from jax import lax
from jax.experimental import pallas as pl
from jax.experimental.pallas import tpu as pltpu
from jax.experimental.pallas import tpu_sc as plsc

UNITS = 26
LOWER = -17.0
UPPER = 8.0
BW = 0.05
NUM_BUCKETS = int((UPPER - LOWER) / BW) + 1
RESIDUE = LOWER - BW
BATCH = 4096

NC = 2
NS = 16
NW = NC * NS
LANES = 16

ROWS_PER_TILE = BATCH // NW
GROUPS = ROWS_PER_TILE // LANES

assert ROWS_PER_TILE * NW == BATCH and GROUPS * LANES == ROWS_PER_TILE


KCOLS = NUM_BUCKETS + 1


def _tables_body(w_ref, b_ref, a_ref):
    r = jnp.maximum(w_ref[...], jnp.float32(0.0))
    ki = lax.broadcasted_iota(jnp.int32, (NUM_BUCKETS, KCOLS), 0)
    ji = lax.broadcasted_iota(jnp.int32, (NUM_BUCKETS, KCOLS), 1)
    tri = jnp.where(ki < ji, jnp.float32(BW), jnp.float32(0.0))
    acc = lax.dot(r, tri, precision=lax.Precision.HIGHEST,
                  preferred_element_type=jnp.float32)
    ones_row = jnp.full((1, KCOLS), 1.0, jnp.float32)
    bb = lax.dot_general(b_ref[...], ones_row,
                         dimension_numbers=(((0,), (0,)), ((), ())),
                         precision=lax.Precision.HIGHEST,
                         preferred_element_type=jnp.float32)
    a_ref[...] = acc + bb + jnp.float32(RESIDUE)


def _sc_body(x_hbm, a_hbm, o_hbm, x_v, a_v, c_v, o_v, sem):
    wid = lax.axis_index("s") * NC + lax.axis_index("c")
    col0 = wid * ROWS_PER_TILE
    cx = pltpu.async_copy(x_hbm.at[:, pl.ds(col0, ROWS_PER_TILE)], x_v, sem)
    ca = pltpu.async_copy(a_hbm, a_v, sem)
    cx.wait()

    clo = jnp.float32(LOWER + 1e-09)
    chi = jnp.float32(UPPER - 1e-09)
    shiftf = jnp.float32(BW - LOWER)
    invbw = jnp.float32(1.0 / BW)
    idxcap = jnp.float32(NUM_BUCKETS - 0.5)
    half = jnp.float32(0.5)
    two = jnp.float32(2.0)
    one = jnp.float32(1.0)

    @plsc.parallel_loop(0, UNITS * GROUPS, unroll=8)
    def bucketize(i):
        u = i >> 3
        off = (i & (GROUPS - 1)) * LANES
        xc = jnp.clip(x_v.at[u][pl.ds(off, LANES)], clo, chi)
        t = (xc + shiftf) * invbw
        idxf = jnp.minimum(t, idxcap).astype(jnp.int32).astype(jnp.float32)
        c_v.at[u][pl.ds(off, LANES)] = idxf + (t - idxf) * half

    ca.wait()

    @plsc.parallel_loop(0, UNITS * GROUPS, unroll=8)
    def combine(i):
        u = i >> 3
        off = (i & (GROUPS - 1)) * LANES
        ucol = jnp.full((LANES,), u, jnp.int32)
        c = c_v.at[u][pl.ds(off, LANES)]
        idx = c.astype(jnp.int32)
        frac2 = c - idx.astype(jnp.float32)
        a0 = plsc.load_gather(a_v, [ucol, idx])
        a1 = plsc.load_gather(a_v, [ucol, idx + 1])
        z = a0 + frac2 * two * (a1 - a0)
        o_v.at[u][pl.ds(off, LANES)] = one / (one + jnp.exp(-z))

    pltpu.sync_copy(o_v, o_hbm.at[:, pl.ds(col0, ROWS_PER_TILE)])


def kernel(x, weights, bias):
    a2 = pl.pallas_call(
        _tables_body,
        out_shape=jax.ShapeDtypeStruct((UNITS, KCOLS), jnp.float32),
        compiler_params=pltpu.CompilerParams(skip_device_barrier=True),
    )(weights.astype(jnp.float32), bias.astype(jnp.float32)[None, :])

    sc = functools.partial(
        pl.kernel,
        out_type=jax.ShapeDtypeStruct((UNITS, BATCH), jnp.float32),
        mesh=plsc.VectorSubcoreMesh(core_axis_name="c", subcore_axis_name="s"),
        scratch_types=[
            pltpu.VMEM((UNITS, ROWS_PER_TILE), jnp.float32),
            pltpu.VMEM((UNITS, KCOLS), jnp.float32),
            pltpu.VMEM((UNITS, ROWS_PER_TILE), jnp.float32),
            pltpu.VMEM((UNITS, ROWS_PER_TILE), jnp.float32),
            pltpu.SemaphoreType.DMA,
        ],
        compiler_params=pltpu.CompilerParams(needs_layout_passes=False,
                                             use_tc_tiling_on_sc=True,
                                             skip_device_barrier=True),
    )(_sc_body)
    return sc(x.T, a2).T
